# Initial kernel scaffold; baseline (speedup 1.0000x reference)
import jax
import jax.numpy as jnp
from jax.experimental import pallas as pl


def kernel(story, query, word_weight, pos_embed):
    raise NotImplementedError("write your pallas kernel here")


# speedup vs baseline: 3.1344x; 3.1344x over previous
import jax
import jax.numpy as jnp
from jax.experimental import pallas as pl
from jax.experimental.pallas import tpu as pltpu
from jax.experimental.pallas import tpu_sc as plsc

_GW = 128
_NSEG = 256


def _postproc_body(raw_ref, idx_ref, pos_ref, emb_ref, mask_ref, sum_ref):
    raw = raw_ref[...]
    pos = pos_ref[...]
    emb = raw * pos[None, :, :]
    emb_ref[...] = emb
    idx = idx_ref[...]
    m = idx != 0
    mask_ref[...] = m
    mf = m.astype(jnp.float32)
    sum_ref[...] = jnp.sum(emb * mf[:, :, None], axis=1)


def _postproc(raw3, seg_idx, pos_embed):
    nseg, w, e = raw3.shape
    return pl.pallas_call(
        _postproc_body,
        grid=(nseg // _NSEG,),
        in_specs=[
            pl.BlockSpec((_NSEG, w, e), lambda i: (i, 0, 0)),
            pl.BlockSpec((_NSEG, w), lambda i: (i, 0)),
            pl.BlockSpec((w, e), lambda i: (0, 0)),
        ],
        out_specs=[
            pl.BlockSpec((_NSEG, w, e), lambda i: (i, 0, 0)),
            pl.BlockSpec((_NSEG, w), lambda i: (i, 0)),
            pl.BlockSpec((_NSEG, e), lambda i: (i, 0)),
        ],
        out_shape=[
            jax.ShapeDtypeStruct((nseg, w, e), jnp.float32),
            jax.ShapeDtypeStruct((nseg, w), jnp.bool_),
            jax.ShapeDtypeStruct((nseg, e), jnp.float32),
        ],
    )(raw3, seg_idx, pos_embed)


def kernel(story, query, word_weight, pos_embed):
    B, S, W = story.shape
    E = word_weight.shape[1]
    n_story = B * S * W
    n_query = B * W

    story_idx = story.reshape(1, n_story)
    query_idx = query.reshape(1, n_query)

    mesh = plsc.VectorSubcoreMesh(core_axis_name="c", subcore_axis_name="s")

    @pl.kernel(
        out_type=[
            jax.ShapeDtypeStruct((n_story, E), jnp.float32),
            jax.ShapeDtypeStruct((n_query, E), jnp.float32),
        ],
        mesh=mesh,
        compiler_params=pltpu.CompilerParams(use_tc_tiling_on_sc=False),
    )
    def gather_kernel(table_hbm, sidx_hbm, qidx_hbm, sout_hbm, qout_hbm):
        def body(i_vmem, o_vmem):
            pltpu.sync_copy(table_hbm.at[i_vmem.at[0]], o_vmem)

        pltpu.emit_pipeline(
            body,
            grid=(n_story // _GW,),
            in_specs=[pl.BlockSpec((1, _GW), lambda i: (0, i))],
            out_specs=[pl.BlockSpec((_GW, E), lambda i: (i, 0))],
            core_axis_name=("c", "s"),
            dimension_semantics=(pltpu.PARALLEL,),
        )(sidx_hbm, sout_hbm)

        pltpu.emit_pipeline(
            body,
            grid=(n_query // _GW,),
            in_specs=[pl.BlockSpec((1, _GW), lambda i: (0, i))],
            out_specs=[pl.BlockSpec((_GW, E), lambda i: (i, 0))],
            core_axis_name=("c", "s"),
            dimension_semantics=(pltpu.PARALLEL,),
        )(qidx_hbm, qout_hbm)

    raw_story, raw_query = gather_kernel(word_weight, story_idx, query_idx)

    s_emb, s_mask, s_sum = _postproc(
        raw_story.reshape(B * S, W, E), story.reshape(B * S, W), pos_embed[:W]
    )
    q_emb, q_mask, q_sum = _postproc(
        raw_query.reshape(B, W, E), query, pos_embed[:W]
    )

    return (
        s_emb.reshape(B, S, W, E),
        q_emb,
        s_mask.reshape(B, S, W),
        q_mask,
        s_sum.reshape(B, S, E),
        q_sum,
    )
